# baseline (speedup 1.0000x reference)
import jax
import jax.numpy as jnp
from jax import lax
from jax.experimental import pallas as pl

_BLKN = 16384


def _body(sig_ref, cb_ref, out_ref):
    x_t = sig_ref[0]
    cb = cb_ref[...]
    c2 = jnp.sum(cb * cb, axis=1)
    cross_t = lax.dot_general(
        cb, x_t, (((1,), (0,)), ((), ())),
        preferred_element_type=jnp.float32)
    s = c2[:, None] - 2.0 * cross_t
    md = jnp.min(s, axis=0, keepdims=True)
    iota = lax.broadcasted_iota(jnp.int32, s.shape, 0)
    first = jnp.min(jnp.where(s == md, iota, 32), axis=0, keepdims=True)
    onehot = (iota == first).astype(jnp.float32)
    out_ref[0] = lax.dot_general(
        cb, onehot, (((0,), (0,)), ((), ())),
        preferred_element_type=jnp.float32)


def _decode_chunk(sig, codebook):
    b, n, d = sig.shape
    k = codebook.shape[0]
    sig_t = jnp.transpose(sig, (0, 2, 1))
    out_t = pl.pallas_call(
        _body,
        grid=(b, n // _BLKN),
        in_specs=[
            pl.BlockSpec((1, d, _BLKN), lambda i, j: (i, 0, j)),
            pl.BlockSpec((k, d), lambda i, j: (0, 0)),
        ],
        out_specs=pl.BlockSpec((1, d, _BLKN), lambda i, j: (i, 0, j)),
        out_shape=jax.ShapeDtypeStruct((b, d, n), jnp.float32),
    )(sig_t, codebook)
    return jnp.transpose(out_t, (0, 2, 1))


def kernel(signal, codebook):
    b = signal.shape[0]
    h = b // 2
    o1 = _decode_chunk(signal[:h], codebook)
    o2 = _decode_chunk(signal[h:], codebook)
    return jnp.concatenate([o1, o2], axis=0)
